# Initial kernel scaffold; baseline (speedup 1.0000x reference)
import jax
import jax.numpy as jnp
from jax.experimental import pallas as pl


def kernel(points, feats, keep, values_weight):
    raise NotImplementedError("write your pallas kernel here")


# speedup vs baseline: 27.3722x; 27.3722x over previous
import functools

import jax
import jax.numpy as jnp
from jax import lax
from jax.experimental import pallas as pl
from jax.experimental.pallas import tpu as pltpu
from jax.experimental.pallas import tpu_sc as plsc

_TOTAL = 262144
_EMBED = 16
_B = _TOTAL * 8
_LANES = 128
_ROWS = _B // _LANES
_NW = 32
_ROWS_PER_W = _ROWS // _NW
_CHUNK = 16
_NCHUNK = _ROWS_PER_W // _CHUNK

_mesh = plsc.VectorSubcoreMesh(core_axis_name="c", subcore_axis_name="s")


@functools.partial(
    pl.kernel,
    out_type=jax.ShapeDtypeStruct((_ROWS, _LANES, _EMBED), jnp.float32),
    mesh=_mesh,
    scratch_types=[
        pltpu.VMEM((_CHUNK, _LANES), jnp.int32),
        pltpu.VMEM((_CHUNK, _LANES, _EMBED), jnp.float32),
        pltpu.SemaphoreType.DMA,
    ],
    compiler_params=pltpu.CompilerParams(use_tc_tiling_on_sc=False),
)
def _sc_gather(idx_hbm, table_hbm, out_hbm, idx_v, rows_v, sem):
    w = lax.axis_index("s") * 2 + lax.axis_index("c")
    w_base = w * _ROWS_PER_W

    @pl.loop(0, _NCHUNK)
    def _chunk(ci):
        r0 = w_base + ci * _CHUNK
        pltpu.sync_copy(idx_hbm.at[pl.ds(r0, _CHUNK)], idx_v)
        handles = [
            pltpu.async_copy(table_hbm.at[idx_v.at[j]], rows_v.at[j], sem)
            for j in range(_CHUNK)
        ]
        for h in handles:
            h.wait()
        pltpu.sync_copy(rows_v, out_hbm.at[pl.ds(r0, _CHUNK)])


def kernel(points, feats, keep, values_weight):
    mask = keep.astype(bool)
    feats_k = jnp.where(mask[:, None], feats, 0)
    points_k = jnp.where(mask[:, None], points, 0.0)
    idx2d = feats_k.reshape(_ROWS, _LANES)
    emb = _sc_gather(idx2d, values_weight)
    return (
        feats_k[None],
        points_k[None],
        values_weight,
        emb.reshape(1, _TOTAL, 8, _EMBED),
    )
